# baseline (speedup 1.0000x reference)
import functools
import math

import jax
import jax.numpy as jnp
import numpy as np
from jax import lax
from jax.experimental import pallas as pl
from jax.experimental.pallas import tpu as pltpu
from jax.experimental.pallas import tpu_sc as plsc

_N = 16384
_S = 256
_R = 256

_NC = 2
_NS = 16
_NW = _NC * _NS
_NH = _N // 2
_C = 32
_CPW = _NH // _NW
_CHUNKS = _CPW // _C


def _bitrev8(x: int) -> int:
    return int("{:08b}".format(x)[::-1], 2)


_BRV = [_bitrev8(i) for i in range(_S)]
_BRV_NP = np.array(_BRV, dtype=np.int32)


def _stage_list():
    stages = []
    k = 2
    while k <= _S:
        s = k // 2
        while s >= 1:
            kp = _bitrev8(k) if k <= 128 else 256
            sp = _bitrev8(s)
            stages.append((kp, sp))
            s //= 2
        k *= 2
    return stages


_STAGES = _stage_list()


def _partner(x, s):
    m0 = _S // (2 * s)
    x4 = x.reshape(m0, 2, s, _R)
    sw = jnp.concatenate([x4[:, 1:2], x4[:, 0:1]], axis=1)
    return sw.reshape(_S, _R)


def _tc_body(coord_ref, stone_nat_ref, stone_scr_ref, brv2_ref, fused_ref,
             kbuf, cbuf):
    flag = coord_ref[0:1, :]
    cy = coord_ref[1:2, :]
    cx = coord_ref[2:3, :]

    sy_n = stone_nat_ref[:, 1:2]
    sx_n = stone_nat_ref[:, 2:3]
    t = jnp.broadcast_to(sy_n - cy, (_S, _R))
    nt = lax.bitcast_convert_type(
        lax.bitcast_convert_type(t, jnp.int32) ^ np.int32(-2**31),
        jnp.float32)
    ang = jnp.arctan2(nt, sx_n - cx) * (180.0 / math.pi)

    sv = stone_scr_ref[:, 0:1]
    sy = stone_scr_ref[:, 1:2]
    sx = stone_scr_ref[:, 2:3]
    dy = sy - cy
    dx = sx - cx
    key = jnp.sqrt(dy * dy + dx * dx)

    valbit = (sv > 0).astype(jnp.int32)
    kbuf[...] = jnp.broadcast_to(key, (_S, _R))
    cbuf[...] = jnp.broadcast_to(brv2_ref[...] + valbit, (_S, _R))
    i = lax.broadcasted_iota(jnp.int32, (_S, 1), 0)

    for kp, sp in _STAGES:
        if sp >= 8 and (kp >= 8 or kp == 256):
            frag = sp if kp == 256 else kp
            for o in range(0, _S, 2 * sp):
                for u in range(0, sp, frag):
                    asc = (u // frag) % 2 == 0 if kp != 256 else True
                    a0 = o + u
                    b0 = o + sp + u
                    ka = kbuf[a0:a0 + frag]
                    kb = kbuf[b0:b0 + frag]
                    ca = cbuf[a0:a0 + frag]
                    cb = cbuf[b0:b0 + frag]
                    a_less = (ka < kb) | ((ka == kb) & (ca < cb))
                    if asc:
                        kbuf[a0:a0 + frag] = jnp.where(a_less, ka, kb)
                        kbuf[b0:b0 + frag] = jnp.where(a_less, kb, ka)
                        cbuf[a0:a0 + frag] = jnp.where(a_less, ca, cb)
                        cbuf[b0:b0 + frag] = jnp.where(a_less, cb, ca)
                    else:
                        kbuf[a0:a0 + frag] = jnp.where(a_less, kb, ka)
                        kbuf[b0:b0 + frag] = jnp.where(a_less, ka, kb)
                        cbuf[a0:a0 + frag] = jnp.where(a_less, cb, ca)
                        cbuf[b0:b0 + frag] = jnp.where(a_less, ca, cb)
        elif sp >= 8:
            j = lax.broadcasted_iota(jnp.int32, (sp, 1), 0)
            dirmask = (j & kp) == 0
            for o in range(0, _S, 2 * sp):
                ka = kbuf[o:o + sp]
                kb = kbuf[o + sp:o + 2 * sp]
                ca = cbuf[o:o + sp]
                cb = cbuf[o + sp:o + 2 * sp]
                a_less = (ka < kb) | ((ka == kb) & (ca < cb))
                keep = a_less == dirmask
                kbuf[o:o + sp] = jnp.where(keep, ka, kb)
                kbuf[o + sp:o + 2 * sp] = jnp.where(keep, kb, ka)
                cbuf[o:o + sp] = jnp.where(keep, ca, cb)
                cbuf[o + sp:o + 2 * sp] = jnp.where(keep, cb, ca)
        else:
            key = kbuf[...]
            code = cbuf[...]
            ok = _partner(key, sp)
            oc = _partner(code, sp)
            tm = ((i & kp) == 0) == ((i & sp) == 0)
            is_less = (key < ok) | ((key == ok) & (code < oc))
            sel = is_less == tm
            kbuf[...] = jnp.where(sel, key, ok)
            cbuf[...] = jnp.where(sel, code, oc)

    live = flag == 0.0
    fused_ref[:, 0:_S] = jnp.transpose(jnp.where(live, cbuf[...], -2))
    fused_ref[:, _S:2 * _S] = lax.bitcast_convert_type(
        jnp.transpose(jnp.where(live, kbuf[...], 0.0)), jnp.int32)
    fused_ref[:, 2 * _S:3 * _S] = lax.bitcast_convert_type(
        jnp.transpose(jnp.where(live, ang, 0.0)), jnp.int32)


_sc_mesh = plsc.VectorSubcoreMesh(core_axis_name="c", subcore_axis_name="s")


def _bitrev4(x: int) -> int:
    return int("{:04b}".format(x)[::-1], 2)


@functools.partial(
    pl.kernel,
    out_type=jax.ShapeDtypeStruct((_NH, 3 * _S), jnp.float32),
    mesh=_sc_mesh,
    scratch_types=[
        pltpu.VMEM((_C, 3 * _S), jnp.int32),
        pltpu.VMEM((_C, 3 * _S), jnp.int32),
        pltpu.VMEM((_C, 3 * _S), jnp.float32),
        pltpu.VMEM((_C, 3 * _S), jnp.float32),
        pltpu.SemaphoreType.DMA,
        pltpu.SemaphoreType.DMA,
        pltpu.SemaphoreType.DMA,
        pltpu.SemaphoreType.DMA,
    ],
    compiler_params=pltpu.CompilerParams(needs_layout_passes=False),
)
def _sc_assemble(fused_hbm, out_hbm, in_v0, in_v1, out_v0, out_v1,
                 si0, si1, so0, so1):
    wid = lax.axis_index("s") * _NC + lax.axis_index("c")
    base = wid * _CPW
    lane = lax.iota(jnp.int32, 16)
    one = jnp.ones((16,), jnp.int32)
    brv4_hi = (((lane & 1) << 3) | ((lane & 2) << 1) | ((lane & 4) >> 1)
               | ((lane & 8) >> 3)) << 4

    def in_slice(ci):
        return fused_hbm.at[pl.ds(base + ci * _C, _C), :]

    def out_slice(ci):
        return out_hbm.at[pl.ds(base + ci * _C, _C), :]

    zero = one * 0
    two = one + one

    def compute(in_v, out_v):
        def c_body(c, carry2):
            cvec = one * c
            for r in range(_S // 16):
                r16 = 16 * r
                pos = brv4_hi + _bitrev4(r)
                raw = in_v[c, r16:r16 + 16]
                dist = plsc.bitcast(in_v[c, _S + r16:_S + r16 + 16],
                                    jnp.float32)
                sidx = lax.shift_right_arithmetic(jnp.maximum(raw, 0), 1)
                val = jnp.where(raw < 0, 0.0,
                                jnp.where((raw & 1) == 1, 1.0, -1.0))
                ang = plsc.bitcast(
                    plsc.load_gather(in_v, [cvec, sidx + 2 * _S]),
                    jnp.float32)
                pos3 = pos * 3
                plsc.store_scatter(out_v, [cvec, pos3], val)
                plsc.store_scatter(out_v, [cvec, pos3 + 1], dist)
                plsc.store_scatter(out_v, [cvec, pos3 + 2], ang)
            return carry2

        lax.fori_loop(0, _C, c_body, 0)

    bufs = ((in_v0, out_v0, si0, so0), (in_v1, out_v1, si1, so1))

    pltpu.async_copy(in_slice(0), in_v0, si0)
    pltpu.async_copy(in_slice(1), in_v1, si1)

    def pair_body(g, carry):
        for half in (0, 1):
            in_v, out_v, si, so = bufs[half]
            ci = 2 * g + half
            pltpu.make_async_copy(in_slice(ci), in_v, si).wait()

            @pl.when(g > 0)
            def _():
                pltpu.make_async_copy(out_v, out_slice(ci), so).wait()

            compute(in_v, out_v)
            pltpu.async_copy(out_v, out_slice(ci), so)

            @pl.when(ci + 2 < _CHUNKS)
            def _():
                pltpu.async_copy(in_slice(ci + 2), in_v, si)
        return carry

    lax.fori_loop(0, _CHUNKS // 2, pair_body, 0)
    pltpu.make_async_copy(out_v0, out_slice(_CHUNKS - 2), so0).wait()
    pltpu.make_async_copy(out_v1, out_slice(_CHUNKS - 1), so1).wait()


def kernel(all_coord_input, stone_coord_input):
    coords_t = all_coord_input.astype(jnp.float32).T
    stones = stone_coord_input.astype(jnp.float32)
    stones_scr = stones[_BRV_NP]
    brv2 = jnp.asarray((2 * _BRV_NP).reshape(_S, 1))
    grid = _NH // _R

    def tc_half(coords_half):
        return pl.pallas_call(
            _tc_body,
            grid=(grid,),
            in_specs=[
                pl.BlockSpec((3, _R), lambda i: (0, i)),
                pl.BlockSpec((_S, 3), lambda i: (0, 0)),
                pl.BlockSpec((_S, 3), lambda i: (0, 0)),
                pl.BlockSpec((_S, 1), lambda i: (0, 0)),
            ],
            out_specs=pl.BlockSpec((_R, 3 * _S), lambda i: (i, 0)),
            out_shape=jax.ShapeDtypeStruct((_NH, 3 * _S), jnp.int32),
            scratch_shapes=[
                pltpu.VMEM((_S, _R), jnp.float32),
                pltpu.VMEM((_S, _R), jnp.int32),
            ],
        )(coords_half, stones, stones_scr, brv2)

    fused0 = tc_half(coords_t[:, :_NH])
    fused1 = tc_half(coords_t[:, _NH:])
    out0 = _sc_assemble(fused0)
    out1 = _sc_assemble(fused1)
    out = jnp.concatenate([out0, out1], axis=0)
    return out.reshape(_N, _S, 3)


# speedup vs baseline: 1.0002x; 1.0002x over previous
import functools
import math

import jax
import jax.numpy as jnp
import numpy as np
from jax import lax
from jax.experimental import pallas as pl
from jax.experimental.pallas import tpu as pltpu
from jax.experimental.pallas import tpu_sc as plsc

_N = 16384
_S = 256
_R = 256

_NC = 2
_NS = 16
_NW = _NC * _NS
_NH = _N // 2
_C = 32
_CPW = _NH // _NW
_CHUNKS = _CPW // _C


def _bitrev8(x: int) -> int:
    return int("{:08b}".format(x)[::-1], 2)


_BRV = [_bitrev8(i) for i in range(_S)]
_BRV_NP = np.array(_BRV, dtype=np.int32)


def _stage_list():
    stages = []
    k = 2
    while k <= _S:
        s = k // 2
        while s >= 1:
            kp = _bitrev8(k) if k <= 128 else 256
            sp = _bitrev8(s)
            stages.append((kp, sp))
            s //= 2
        k *= 2
    return stages


_STAGES = _stage_list()


def _partner(x, s):
    m0 = _S // (2 * s)
    x4 = x.reshape(m0, 2, s, _R)
    sw = jnp.concatenate([x4[:, 1:2], x4[:, 0:1]], axis=1)
    return sw.reshape(_S, _R)


def _tc_body(coord_ref, stone_nat_ref, stone_scr_ref, brv2_ref, fused_ref,
             kbuf, cbuf):
    flag = coord_ref[0:1, :]
    cy = coord_ref[1:2, :]
    cx = coord_ref[2:3, :]

    sy_n = stone_nat_ref[:, 1:2]
    sx_n = stone_nat_ref[:, 2:3]
    t = jnp.broadcast_to(sy_n - cy, (_S, _R))
    nt = lax.bitcast_convert_type(
        lax.bitcast_convert_type(t, jnp.int32) ^ np.int32(-2**31),
        jnp.float32)
    ang = jnp.arctan2(nt, sx_n - cx) * (180.0 / math.pi)

    sv = stone_scr_ref[:, 0:1]
    sy = stone_scr_ref[:, 1:2]
    sx = stone_scr_ref[:, 2:3]
    dy = sy - cy
    dx = sx - cx
    key = jnp.sqrt(dy * dy + dx * dx)

    valbit = (sv > 0).astype(jnp.int32)
    kbuf[...] = jnp.broadcast_to(key, (_S, _R))
    cbuf[...] = jnp.broadcast_to(brv2_ref[...] + valbit, (_S, _R))
    i = lax.broadcasted_iota(jnp.int32, (_S, 1), 0)

    for kp, sp in _STAGES:
        if sp >= 8 and (kp >= 8 or kp == 256):
            frag = sp if kp == 256 else kp
            for o in range(0, _S, 2 * sp):
                for u in range(0, sp, frag):
                    asc = (u // frag) % 2 == 0 if kp != 256 else True
                    a0 = o + u
                    b0 = o + sp + u
                    ka = kbuf[a0:a0 + frag]
                    kb = kbuf[b0:b0 + frag]
                    ca = cbuf[a0:a0 + frag]
                    cb = cbuf[b0:b0 + frag]
                    a_less = (ka < kb) | ((ka == kb) & (ca < cb))
                    if asc:
                        kbuf[a0:a0 + frag] = jnp.where(a_less, ka, kb)
                        kbuf[b0:b0 + frag] = jnp.where(a_less, kb, ka)
                        cbuf[a0:a0 + frag] = jnp.where(a_less, ca, cb)
                        cbuf[b0:b0 + frag] = jnp.where(a_less, cb, ca)
                    else:
                        kbuf[a0:a0 + frag] = jnp.where(a_less, kb, ka)
                        kbuf[b0:b0 + frag] = jnp.where(a_less, ka, kb)
                        cbuf[a0:a0 + frag] = jnp.where(a_less, cb, ca)
                        cbuf[b0:b0 + frag] = jnp.where(a_less, ca, cb)
        elif sp >= 8:
            j = lax.broadcasted_iota(jnp.int32, (sp, 1), 0)
            dirmask = (j & kp) == 0
            for o in range(0, _S, 2 * sp):
                ka = kbuf[o:o + sp]
                kb = kbuf[o + sp:o + 2 * sp]
                ca = cbuf[o:o + sp]
                cb = cbuf[o + sp:o + 2 * sp]
                a_less = (ka < kb) | ((ka == kb) & (ca < cb))
                keep = a_less == dirmask
                kbuf[o:o + sp] = jnp.where(keep, ka, kb)
                kbuf[o + sp:o + 2 * sp] = jnp.where(keep, kb, ka)
                cbuf[o:o + sp] = jnp.where(keep, ca, cb)
                cbuf[o + sp:o + 2 * sp] = jnp.where(keep, cb, ca)
        else:
            key = kbuf[...]
            code = cbuf[...]
            ok = _partner(key, sp)
            oc = _partner(code, sp)
            tm = ((i & kp) == 0) == ((i & sp) == 0)
            is_less = (key < ok) | ((key == ok) & (code < oc))
            sel = is_less == tm
            kbuf[...] = jnp.where(sel, key, ok)
            cbuf[...] = jnp.where(sel, code, oc)

    live = flag == 0.0
    fused_ref[:, 0:_S] = jnp.transpose(jnp.where(live, cbuf[...], -2))
    fused_ref[:, _S:2 * _S] = lax.bitcast_convert_type(
        jnp.transpose(jnp.where(live, kbuf[...], 0.0)), jnp.int32)
    fused_ref[:, 2 * _S:3 * _S] = lax.bitcast_convert_type(
        jnp.transpose(jnp.where(live, ang, 0.0)), jnp.int32)


_sc_mesh = plsc.VectorSubcoreMesh(core_axis_name="c", subcore_axis_name="s")


def _bitrev4(x: int) -> int:
    return int("{:04b}".format(x)[::-1], 2)


@functools.partial(
    pl.kernel,
    out_type=jax.ShapeDtypeStruct((_NH, 3 * _S), jnp.float32),
    mesh=_sc_mesh,
    scratch_types=[
        pltpu.VMEM((_C, 3 * _S), jnp.int32),
        pltpu.VMEM((_C, 3 * _S), jnp.int32),
        pltpu.VMEM((_C, 3 * _S), jnp.float32),
        pltpu.VMEM((_C, 3 * _S), jnp.float32),
        pltpu.SemaphoreType.DMA,
        pltpu.SemaphoreType.DMA,
        pltpu.SemaphoreType.DMA,
        pltpu.SemaphoreType.DMA,
    ],
    compiler_params=pltpu.CompilerParams(needs_layout_passes=False),
)
def _sc_assemble(fused_hbm, out_hbm, in_v0, in_v1, out_v0, out_v1,
                 si0, si1, so0, so1):
    wid = lax.axis_index("s") * _NC + lax.axis_index("c")
    base = wid * _CPW
    lane = lax.iota(jnp.int32, 16)
    one = jnp.ones((16,), jnp.int32)
    brv4_hi = (((lane & 1) << 3) | ((lane & 2) << 1) | ((lane & 4) >> 1)
               | ((lane & 8) >> 3)) << 4

    def in_slice(ci):
        return fused_hbm.at[pl.ds(base + ci * _C, _C), :]

    def out_slice(ci):
        return out_hbm.at[pl.ds(base + ci * _C, _C), :]

    def compute(in_v, out_v):
        def c_body(c, carry2):
            cvec = one * c
            for r in range(_S // 16):
                r16 = 16 * r
                pos = brv4_hi + _bitrev4(r)
                raw = in_v[c, r16:r16 + 16]
                dist = plsc.bitcast(in_v[c, _S + r16:_S + r16 + 16],
                                    jnp.float32)
                sidx = lax.shift_right_arithmetic(jnp.maximum(raw, 0), 1)
                val = jnp.where(raw < 0, 0.0,
                                jnp.where((raw & 1) == 1, 1.0, -1.0))
                ang = plsc.bitcast(
                    plsc.load_gather(in_v, [cvec, sidx + 2 * _S]),
                    jnp.float32)
                pos3 = pos * 3
                plsc.store_scatter(out_v, [cvec, pos3], val)
                plsc.store_scatter(out_v, [cvec, pos3 + 1], dist)
                plsc.store_scatter(out_v, [cvec, pos3 + 2], ang)
            return carry2

        lax.fori_loop(0, _C, c_body, 0)

    bufs = ((in_v0, out_v0, si0, so0), (in_v1, out_v1, si1, so1))

    pltpu.async_copy(in_slice(0), in_v0, si0)
    pltpu.async_copy(in_slice(1), in_v1, si1)

    def pair_body(g, carry):
        for half in (0, 1):
            in_v, out_v, si, so = bufs[half]
            ci = 2 * g + half
            pltpu.make_async_copy(in_slice(ci), in_v, si).wait()

            @pl.when(g > 0)
            def _():
                pltpu.make_async_copy(out_v, out_slice(ci), so).wait()

            compute(in_v, out_v)
            pltpu.async_copy(out_v, out_slice(ci), so)

            @pl.when(ci + 2 < _CHUNKS)
            def _():
                pltpu.async_copy(in_slice(ci + 2), in_v, si)
        return carry

    lax.fori_loop(0, _CHUNKS // 2, pair_body, 0)
    pltpu.make_async_copy(out_v0, out_slice(_CHUNKS - 2), so0).wait()
    pltpu.make_async_copy(out_v1, out_slice(_CHUNKS - 1), so1).wait()


def kernel(all_coord_input, stone_coord_input):
    coords_t = all_coord_input.astype(jnp.float32).T
    stones = stone_coord_input.astype(jnp.float32)
    stones_scr = stones[_BRV_NP]
    brv2 = jnp.asarray((2 * _BRV_NP).reshape(_S, 1))
    grid = _NH // _R

    def tc_half(coords_half):
        return pl.pallas_call(
            _tc_body,
            grid=(grid,),
            in_specs=[
                pl.BlockSpec((3, _R), lambda i: (0, i)),
                pl.BlockSpec((_S, 3), lambda i: (0, 0)),
                pl.BlockSpec((_S, 3), lambda i: (0, 0)),
                pl.BlockSpec((_S, 1), lambda i: (0, 0)),
            ],
            out_specs=pl.BlockSpec((_R, 3 * _S), lambda i: (i, 0)),
            out_shape=jax.ShapeDtypeStruct((_NH, 3 * _S), jnp.int32),
            scratch_shapes=[
                pltpu.VMEM((_S, _R), jnp.float32),
                pltpu.VMEM((_S, _R), jnp.int32),
            ],
        )(coords_half, stones, stones_scr, brv2)

    fused0 = tc_half(coords_t[:, :_NH])
    fused1 = tc_half(coords_t[:, _NH:])
    out0 = _sc_assemble(fused0)
    out1 = _sc_assemble(fused1)
    out = jnp.concatenate([out0, out1], axis=0)
    return out.reshape(_N, _S, 3)
